# Initial kernel scaffold; baseline (speedup 1.0000x reference)
import jax
import jax.numpy as jnp
from jax.experimental import pallas as pl


def kernel(hidden_states, W, b):
    raise NotImplementedError("write your pallas kernel here")


# speedup vs baseline: 1.2026x; 1.2026x over previous
import jax
import jax.numpy as jnp
import numpy as np
from jax.experimental import pallas as pl

_G = 32
_V = 320
_T = 256

_ROT_A = (13, 15, 26, 6)
_ROT_B = (17, 29, 16, 24)
_KS1 = np.uint32(42)
_KS2 = np.uint32(42 ^ 0x1BD11BDA)


def _threefry_bits(idx):
    x0 = jnp.zeros_like(idx)
    x1 = idx + _KS1

    def rounds(x0, x1, rots):
        for r in rots:
            x0 = x0 + x1
            x1 = (x1 << np.uint32(r)) | (x1 >> np.uint32(32 - r))
            x1 = x1 ^ x0
        return x0, x1

    x0, x1 = rounds(x0, x1, _ROT_A)
    x0 = x0 + _KS1
    x1 = x1 + (_KS2 + np.uint32(1))
    x0, x1 = rounds(x0, x1, _ROT_B)
    x0 = x0 + _KS2
    x1 = x1 + np.uint32(2)
    x0, x1 = rounds(x0, x1, _ROT_A)
    x1 = x1 + (_KS1 + np.uint32(3))
    x0, x1 = rounds(x0, x1, _ROT_B)
    x0 = x0 + _KS1
    x1 = x1 + (_KS2 + np.uint32(4))
    x0, x1 = rounds(x0, x1, _ROT_A)
    x0 = x0 + _KS2
    x1 = x1 + np.uint32(5)
    return x0 ^ x1


def _gumbel_from_bits(bits):
    fb = (bits >> np.uint32(9)) | np.uint32(0x3F800000)
    floats = jax.lax.bitcast_convert_type(fb, jnp.float32) - np.float32(1.0)
    tiny = np.float32(np.finfo(np.float32).tiny)
    u = jnp.maximum(tiny, floats * (np.float32(1.0) - tiny) + tiny)
    return -jnp.log(-jnp.log(u))


def _body(hs_ref, w_ref, b_ref, out_ref):
    t0 = pl.program_id(0) * _T
    x = hs_ref[...]
    row = jax.lax.broadcasted_iota(jnp.uint32, (_T, _V), 0)
    col = jax.lax.broadcasted_iota(jnp.uint32, (_T, _V), 1)
    base = (jnp.uint32(t0) + row) * np.uint32(_G)
    vidx = jax.lax.broadcasted_iota(jnp.int32, (_T, _V), 1)
    for g in range(_G):
        xg = jnp.dot(x, w_ref[g], preferred_element_type=jnp.float32)
        xg = xg + b_ref[g]
        idx = (base + np.uint32(g)) * np.uint32(_V) + col
        y = xg + _gumbel_from_bits(_threefry_bits(idx))
        am = jnp.argmax(y, axis=1).astype(jnp.int32)
        out_ref[:, g, :] = (vidx == am[:, None]).astype(jnp.float32)


def kernel(hidden_states, W, b):
    B, S, H = hidden_states.shape
    n_tok = B * S
    hs = hidden_states.reshape(n_tok, H)
    w3 = W.reshape(H, _G, _V).transpose(1, 0, 2)
    b3 = b.reshape(_G, 1, _V)
    out = pl.pallas_call(
        _body,
        grid=(n_tok // _T,),
        in_specs=[
            pl.BlockSpec((_T, H), lambda i: (i, 0)),
            pl.BlockSpec((_G, H, _V), lambda i: (0, 0, 0)),
            pl.BlockSpec((_G, 1, _V), lambda i: (0, 0, 0)),
        ],
        out_specs=pl.BlockSpec((_T, _G, _V), lambda i: (i, 0, 0)),
        out_shape=jax.ShapeDtypeStruct((n_tok, _G, _V), jnp.float32),
    )(hs, w3, b3)
    return out.reshape(n_tok * _G, _V)


# speedup vs baseline: 1.2283x; 1.0214x over previous
import jax
import jax.numpy as jnp
import numpy as np
from jax.experimental import pallas as pl

_G = 32
_V = 320
_T = 256

_ROT_A = (13, 15, 26, 6)
_ROT_B = (17, 29, 16, 24)
_KS1 = np.uint32(42)
_KS2 = np.uint32(42 ^ 0x1BD11BDA)


def _threefry_bits(idx):
    x0 = jnp.zeros_like(idx)
    x1 = idx + _KS1

    def rounds(x0, x1, rots):
        for r in rots:
            x0 = x0 + x1
            x1 = (x1 << np.uint32(r)) | (x1 >> np.uint32(32 - r))
            x1 = x1 ^ x0
        return x0, x1

    x0, x1 = rounds(x0, x1, _ROT_A)
    x0 = x0 + _KS1
    x1 = x1 + (_KS2 + np.uint32(1))
    x0, x1 = rounds(x0, x1, _ROT_B)
    x0 = x0 + _KS2
    x1 = x1 + np.uint32(2)
    x0, x1 = rounds(x0, x1, _ROT_A)
    x1 = x1 + (_KS1 + np.uint32(3))
    x0, x1 = rounds(x0, x1, _ROT_B)
    x0 = x0 + _KS1
    x1 = x1 + (_KS2 + np.uint32(4))
    x0, x1 = rounds(x0, x1, _ROT_A)
    x0 = x0 + _KS2
    x1 = x1 + np.uint32(5)
    return x0 ^ x1


def _gumbel_from_bits(bits):
    fb = (bits >> np.uint32(9)) | np.uint32(0x3F800000)
    floats = jax.lax.bitcast_convert_type(fb, jnp.float32) - np.float32(1.0)
    u = floats + np.float32(np.finfo(np.float32).tiny)
    return -jnp.log(-jnp.log(u))


def _body(hs_ref, w_ref, b_ref, out_ref):
    t0 = pl.program_id(0) * _T
    x = hs_ref[...]
    row = jax.lax.broadcasted_iota(jnp.uint32, (_T, _V), 0)
    col = jax.lax.broadcasted_iota(jnp.uint32, (_T, _V), 1)
    flat = (jnp.uint32(t0) + row) * np.uint32(_G * _V) + col
    one = jnp.float32(1.0)
    zero = jnp.float32(0.0)
    for g in range(_G):
        xg = jnp.dot(x, w_ref[g], preferred_element_type=jnp.float32)
        xg = xg + b_ref[g]
        y = xg + _gumbel_from_bits(_threefry_bits(flat + np.uint32(g * _V)))
        m = jnp.max(y, axis=1)
        out_ref[:, g, :] = jnp.where(y == m[:, None], one, zero)


def kernel(hidden_states, W, b):
    B, S, H = hidden_states.shape
    n_tok = B * S
    hs = hidden_states.reshape(n_tok, H)
    w3 = W.reshape(H, _G, _V).transpose(1, 0, 2)
    b3 = b.reshape(_G, 1, _V)
    out = pl.pallas_call(
        _body,
        grid=(n_tok // _T,),
        in_specs=[
            pl.BlockSpec((_T, H), lambda i: (i, 0)),
            pl.BlockSpec((_G, H, _V), lambda i: (0, 0, 0)),
            pl.BlockSpec((_G, 1, _V), lambda i: (0, 0, 0)),
        ],
        out_specs=pl.BlockSpec((_T, _G, _V), lambda i: (i, 0, 0)),
        out_shape=jax.ShapeDtypeStruct((n_tok, _G, _V), jnp.float32),
    )(hs, w3, b3)
    return out.reshape(n_tok * _G, _V)
